# baseline (speedup 1.0000x reference)
import jax
import jax.numpy as jnp
from jax import lax
from jax.experimental import pallas as pl
from jax.experimental.pallas import tpu as pltpu

_B = 16384
_COND_DIM = 512
_HIDDEN = 1024
_NUM_BINS = 128
_OUT_DIM = 3 * _NUM_BINS + 1
_OUT_PAD = 512
_TAIL = 5.0
_TILE = 2048
_SCHUNK = 512
_NSTEP = _B // _TILE

_MBW = 1e-3
_MBH = 1e-3
_MDER = 1e-3


def _softmax(a):
    m = jnp.max(a, axis=-1, keepdims=True)
    e = jnp.exp(a - m)
    return e / jnp.sum(e, axis=-1, keepdims=True)


def _spline_chunk(p, xq, tri):
    f32 = jnp.float32
    un_w = p[:, :_NUM_BINS]
    un_h = p[:, _NUM_BINS:2 * _NUM_BINS]
    un_d = p[:, 2 * _NUM_BINS:3 * _NUM_BINS]
    un_d_last = p[:, 3 * _NUM_BINS:3 * _NUM_BINS + 1]

    left, right, bottom, top = -_TAIL, _TAIL, -_TAIL, _TAIL
    widths = _MBW + (right - left - _NUM_BINS * _MBW) * _softmax(un_w)
    heights = _MBH + (top - bottom - _NUM_BINS * _MBH) * _softmax(un_h)

    cw = jnp.dot(widths, tri, preferred_element_type=f32,
                 precision=lax.Precision.HIGHEST)

    d_all = _MDER + jax.nn.softplus(un_d)
    d_last = _MDER + jax.nn.softplus(un_d_last)

    xc = jnp.clip(xq, left, right)

    cmp = xc >= (left + cw)
    iota = lax.broadcasted_iota(jnp.int32, (1, _NUM_BINS), 1)
    mask_lt = jnp.logical_and(cmp, iota < (_NUM_BINS - 1)).astype(f32)
    nlt = jnp.sum(mask_lt, axis=-1, keepdims=True)
    iota_f = iota.astype(f32)
    onehot = (iota_f == nlt).astype(f32)
    onehot_next = (iota_f == (nlt + 1.0)).astype(f32)

    x0 = left + jnp.sum(widths * mask_lt, axis=-1, keepdims=True)
    y0 = bottom + jnp.sum(heights * mask_lt, axis=-1, keepdims=True)
    w = jnp.sum(widths * onehot, axis=-1, keepdims=True)
    hh = jnp.sum(heights * onehot, axis=-1, keepdims=True)
    d0 = jnp.sum(d_all * onehot, axis=-1, keepdims=True)
    is_last = (nlt == (_NUM_BINS - 1)).astype(f32)
    d1 = jnp.sum(d_all * onehot_next, axis=-1, keepdims=True) + d_last * is_last

    delta = hh / w
    theta = (xc - x0) / w
    omt = 1.0 - theta
    num = hh * (delta * theta * theta + d0 * theta * omt)
    den = delta + (d0 + d1 - 2.0 * delta) * theta * omt
    y = y0 + num / den
    der_num = delta * delta * (d1 * theta * theta
                               + 2.0 * delta * theta * omt
                               + d0 * omt * omt)
    der_den = den * den
    logdet = (jnp.log(jnp.maximum(der_num, 1e-12))
              - jnp.log(jnp.maximum(der_den, 1e-12)))

    outside = jnp.logical_or(xq < left, xq > right)
    return jnp.where(outside, xq, y), jnp.where(outside, 0.0, logdet)


def _fused_kernel(x_ref, cond_ref, w0_ref, b0_ref, w1_ref, b1_ref,
                  w2_ref, b2_ref, y_ref, ld_ref, p_scr_ref):
    f32 = jnp.float32
    i = pl.program_id(0)

    @pl.when(i < _NSTEP)
    def _matmul_phase():
        c = cond_ref[:]
        h = jnp.maximum(jnp.dot(c, w0_ref[:], preferred_element_type=f32) + b0_ref[:], 0.0)
        h = jnp.maximum(jnp.dot(h, w1_ref[:], preferred_element_type=f32) + b1_ref[:], 0.0)
        p_scr_ref[lax.rem(i, 2)] = jnp.dot(h, w2_ref[:], preferred_element_type=f32) + b2_ref[:]

    @pl.when(i > 0)
    def _spline_phase():
        r_i = lax.broadcasted_iota(jnp.int32, (_NUM_BINS, _NUM_BINS), 0)
        c_i = lax.broadcasted_iota(jnp.int32, (_NUM_BINS, _NUM_BINS), 1)
        tri = (r_i <= c_i).astype(f32)
        prev = lax.rem(i + 1, 2)
        for j in range(_TILE // _SCHUNK):
            sl = slice(j * _SCHUNK, (j + 1) * _SCHUNK)
            y, ld = _spline_chunk(p_scr_ref[prev, sl], x_ref[sl], tri)
            y_ref[sl] = y
            ld_ref[sl] = ld


@jax.jit
def kernel(x, cond, W0, b0, W1, b1, W2, b2):
    x2 = x.reshape(_B, 1)
    W2p = jnp.pad(W2, ((0, 0), (0, _OUT_PAD - _OUT_DIM)))
    b2p = jnp.pad(b2, (0, _OUT_PAD - _OUT_DIM)).reshape(1, _OUT_PAD)
    b0r = b0.reshape(1, _HIDDEN)
    b1r = b1.reshape(1, _HIDDEN)
    grid = (_NSTEP + 1,)
    y2, ld2 = pl.pallas_call(
        _fused_kernel,
        grid=grid,
        in_specs=[
            pl.BlockSpec((_TILE, 1), lambda i: (jnp.maximum(i - 1, 0), 0)),
            pl.BlockSpec((_TILE, _COND_DIM), lambda i: (jnp.minimum(i, _NSTEP - 1), 0)),
            pl.BlockSpec((_COND_DIM, _HIDDEN), lambda i: (0, 0)),
            pl.BlockSpec((1, _HIDDEN), lambda i: (0, 0)),
            pl.BlockSpec((_HIDDEN, _HIDDEN), lambda i: (0, 0)),
            pl.BlockSpec((1, _HIDDEN), lambda i: (0, 0)),
            pl.BlockSpec((_HIDDEN, _OUT_PAD), lambda i: (0, 0)),
            pl.BlockSpec((1, _OUT_PAD), lambda i: (0, 0)),
        ],
        out_specs=[
            pl.BlockSpec((_TILE, 1), lambda i: (jnp.maximum(i - 1, 0), 0)),
            pl.BlockSpec((_TILE, 1), lambda i: (jnp.maximum(i - 1, 0), 0)),
        ],
        out_shape=[
            jax.ShapeDtypeStruct((_B, 1), jnp.float32),
            jax.ShapeDtypeStruct((_B, 1), jnp.float32),
        ],
        scratch_shapes=[pltpu.VMEM((2, _TILE, _OUT_PAD), jnp.float32)],
        compiler_params=pltpu.CompilerParams(
            dimension_semantics=("arbitrary",),
        ),
    )(x2, cond, W0, b0r, W1, b1r, W2p, b2p)
    return y2.reshape(_B), ld2.reshape(_B)


# speedup vs baseline: 1.0057x; 1.0057x over previous
import jax
import jax.numpy as jnp
from jax import lax
from jax.experimental import pallas as pl
from jax.experimental.pallas import tpu as pltpu

_B = 16384
_COND_DIM = 512
_HIDDEN = 1024
_NUM_BINS = 128
_OUT_DIM = 3 * _NUM_BINS + 1
_OUT_PAD = 512
_TAIL = 5.0
_TILE = 2048
_SCHUNK = 2048
_NSTEP = _B // _TILE

_MBW = 1e-3
_MBH = 1e-3
_MDER = 1e-3


def _softmax(a):
    m = jnp.max(a, axis=-1, keepdims=True)
    e = jnp.exp(a - m)
    return e / jnp.sum(e, axis=-1, keepdims=True)


def _spline_chunk(p, xq, tri):
    f32 = jnp.float32
    un_w = p[:, :_NUM_BINS]
    un_h = p[:, _NUM_BINS:2 * _NUM_BINS]
    un_d = p[:, 2 * _NUM_BINS:3 * _NUM_BINS]
    un_d_last = p[:, 3 * _NUM_BINS:3 * _NUM_BINS + 1]

    left, right, bottom, top = -_TAIL, _TAIL, -_TAIL, _TAIL
    widths = _MBW + (right - left - _NUM_BINS * _MBW) * _softmax(un_w)
    heights = _MBH + (top - bottom - _NUM_BINS * _MBH) * _softmax(un_h)

    cw = jnp.dot(widths, tri, preferred_element_type=f32,
                 precision=lax.Precision.HIGHEST)

    d_all = _MDER + jax.nn.softplus(un_d)
    d_last = _MDER + jax.nn.softplus(un_d_last)

    xc = jnp.clip(xq, left, right)

    cmp = xc >= (left + cw)
    iota = lax.broadcasted_iota(jnp.int32, (1, _NUM_BINS), 1)
    mask_lt = jnp.logical_and(cmp, iota < (_NUM_BINS - 1)).astype(f32)
    nlt = jnp.sum(mask_lt, axis=-1, keepdims=True)
    iota_f = iota.astype(f32)
    onehot = (iota_f == nlt).astype(f32)
    onehot_next = (iota_f == (nlt + 1.0)).astype(f32)

    x0 = left + jnp.sum(widths * mask_lt, axis=-1, keepdims=True)
    y0 = bottom + jnp.sum(heights * mask_lt, axis=-1, keepdims=True)
    w = jnp.sum(widths * onehot, axis=-1, keepdims=True)
    hh = jnp.sum(heights * onehot, axis=-1, keepdims=True)
    d0 = jnp.sum(d_all * onehot, axis=-1, keepdims=True)
    is_last = (nlt == (_NUM_BINS - 1)).astype(f32)
    d1 = jnp.sum(d_all * onehot_next, axis=-1, keepdims=True) + d_last * is_last

    delta = hh / w
    theta = (xc - x0) / w
    omt = 1.0 - theta
    num = hh * (delta * theta * theta + d0 * theta * omt)
    den = delta + (d0 + d1 - 2.0 * delta) * theta * omt
    y = y0 + num / den
    der_num = delta * delta * (d1 * theta * theta
                               + 2.0 * delta * theta * omt
                               + d0 * omt * omt)
    der_den = den * den
    logdet = (jnp.log(jnp.maximum(der_num, 1e-12))
              - jnp.log(jnp.maximum(der_den, 1e-12)))

    outside = jnp.logical_or(xq < left, xq > right)
    return jnp.where(outside, xq, y), jnp.where(outside, 0.0, logdet)


def _step(x_ref, cond_ref, w0_ref, b0_ref, w1_ref, b1_ref, w2_ref, b2_ref,
          y_ref, ld_ref, dst_ref, src_ref):
    f32 = jnp.float32
    c = cond_ref[:]
    h = jnp.maximum(jnp.dot(c, w0_ref[:], preferred_element_type=f32) + b0_ref[:], 0.0)
    h = jnp.maximum(jnp.dot(h, w1_ref[:], preferred_element_type=f32) + b1_ref[:], 0.0)
    dst_ref[:] = jnp.dot(h, w2_ref[:], preferred_element_type=f32) + b2_ref[:]

    r_i = lax.broadcasted_iota(jnp.int32, (_NUM_BINS, _NUM_BINS), 0)
    c_i = lax.broadcasted_iota(jnp.int32, (_NUM_BINS, _NUM_BINS), 1)
    tri = (r_i <= c_i).astype(f32)

    for j in range(_TILE // _SCHUNK):
        sl = slice(j * _SCHUNK, (j + 1) * _SCHUNK)
        y, ld = _spline_chunk(src_ref[sl], x_ref[sl], tri)
        y_ref[sl] = y
        ld_ref[sl] = ld


def _fused_kernel(x_ref, cond_ref, w0_ref, b0_ref, w1_ref, b1_ref,
                  w2_ref, b2_ref, y_ref, ld_ref, pa_ref, pb_ref):
    i = pl.program_id(0)
    args = (x_ref, cond_ref, w0_ref, b0_ref, w1_ref, b1_ref, w2_ref,
            b2_ref, y_ref, ld_ref)

    @pl.when(lax.rem(i, 2) == 0)
    def _even():
        _step(*args, pa_ref, pb_ref)

    @pl.when(lax.rem(i, 2) == 1)
    def _odd():
        _step(*args, pb_ref, pa_ref)


@jax.jit
def kernel(x, cond, W0, b0, W1, b1, W2, b2):
    x2 = x.reshape(_B, 1)
    W2p = jnp.pad(W2, ((0, 0), (0, _OUT_PAD - _OUT_DIM)))
    b2p = jnp.pad(b2, (0, _OUT_PAD - _OUT_DIM)).reshape(1, _OUT_PAD)
    b0r = b0.reshape(1, _HIDDEN)
    b1r = b1.reshape(1, _HIDDEN)
    grid = (_NSTEP + 1,)
    y2, ld2 = pl.pallas_call(
        _fused_kernel,
        grid=grid,
        in_specs=[
            pl.BlockSpec((_TILE, 1), lambda i: (jnp.maximum(i - 1, 0), 0)),
            pl.BlockSpec((_TILE, _COND_DIM), lambda i: (jnp.minimum(i, _NSTEP - 1), 0)),
            pl.BlockSpec((_COND_DIM, _HIDDEN), lambda i: (0, 0)),
            pl.BlockSpec((1, _HIDDEN), lambda i: (0, 0)),
            pl.BlockSpec((_HIDDEN, _HIDDEN), lambda i: (0, 0)),
            pl.BlockSpec((1, _HIDDEN), lambda i: (0, 0)),
            pl.BlockSpec((_HIDDEN, _OUT_PAD), lambda i: (0, 0)),
            pl.BlockSpec((1, _OUT_PAD), lambda i: (0, 0)),
        ],
        out_specs=[
            pl.BlockSpec((_TILE, 1), lambda i: (jnp.maximum(i - 1, 0), 0)),
            pl.BlockSpec((_TILE, 1), lambda i: (jnp.maximum(i - 1, 0), 0)),
        ],
        out_shape=[
            jax.ShapeDtypeStruct((_B, 1), jnp.float32),
            jax.ShapeDtypeStruct((_B, 1), jnp.float32),
        ],
        scratch_shapes=[
            pltpu.VMEM((_TILE, _OUT_PAD), jnp.float32),
            pltpu.VMEM((_TILE, _OUT_PAD), jnp.float32),
        ],
        compiler_params=pltpu.CompilerParams(
            dimension_semantics=("arbitrary",),
        ),
    )(x2, cond, W0, b0r, W1, b1r, W2p, b2p)
    return y2.reshape(_B), ld2.reshape(_B)


# speedup vs baseline: 1.1244x; 1.1180x over previous
import jax
import jax.numpy as jnp
from jax import lax
from jax.experimental import pallas as pl
from jax.experimental.pallas import tpu as pltpu

_B = 16384
_COND_DIM = 512
_HIDDEN = 1024
_NUM_BINS = 128
_OUT_DIM = 3 * _NUM_BINS + 1
_OUT_PAD = 512
_TAIL = 5.0
_TILE = 2048
_SCHUNK = 2048
_NSTEP = _B // _TILE

_MBW = 1e-3
_MBH = 1e-3
_MDER = 1e-3


def _softmax(a):
    m = jnp.max(a, axis=-1, keepdims=True)
    e = jnp.exp(a - m)
    return e / jnp.sum(e, axis=-1, keepdims=True)


def _spline_chunk(p, xq, tri):
    f32 = jnp.float32
    un_w = p[:, :_NUM_BINS]
    un_h = p[:, _NUM_BINS:2 * _NUM_BINS]
    un_d = p[:, 2 * _NUM_BINS:3 * _NUM_BINS]
    un_d_last = p[:, 3 * _NUM_BINS:3 * _NUM_BINS + 1]

    left, right, bottom, top = -_TAIL, _TAIL, -_TAIL, _TAIL
    widths = _MBW + (right - left - _NUM_BINS * _MBW) * _softmax(un_w)
    heights = _MBH + (top - bottom - _NUM_BINS * _MBH) * _softmax(un_h)

    cw = jnp.dot(widths, tri, preferred_element_type=f32,
                 precision=lax.Precision.HIGHEST)

    d_all = _MDER + jax.nn.softplus(un_d)
    d_last = _MDER + jax.nn.softplus(un_d_last)

    xc = jnp.clip(xq, left, right)

    cmp = xc >= (left + cw)
    iota = lax.broadcasted_iota(jnp.int32, (1, _NUM_BINS), 1)
    mask_lt = jnp.logical_and(cmp, iota < (_NUM_BINS - 1)).astype(f32)
    nlt = jnp.sum(mask_lt, axis=-1, keepdims=True)
    iota_f = iota.astype(f32)
    onehot = (iota_f == nlt).astype(f32)
    onehot_next = (iota_f == (nlt + 1.0)).astype(f32)

    x0 = left + jnp.sum(widths * mask_lt, axis=-1, keepdims=True)
    y0 = bottom + jnp.sum(heights * mask_lt, axis=-1, keepdims=True)
    w = jnp.sum(widths * onehot, axis=-1, keepdims=True)
    hh = jnp.sum(heights * onehot, axis=-1, keepdims=True)
    d0 = jnp.sum(d_all * onehot, axis=-1, keepdims=True)
    is_last = (nlt == (_NUM_BINS - 1)).astype(f32)
    d1 = jnp.sum(d_all * onehot_next, axis=-1, keepdims=True) + d_last * is_last

    delta = hh / w
    theta = (xc - x0) / w
    omt = 1.0 - theta
    num = hh * (delta * theta * theta + d0 * theta * omt)
    den = delta + (d0 + d1 - 2.0 * delta) * theta * omt
    y = y0 + num / den
    der_num = delta * delta * (d1 * theta * theta
                               + 2.0 * delta * theta * omt
                               + d0 * omt * omt)
    der_den = den * den
    logdet = (jnp.log(jnp.maximum(der_num, 1e-12))
              - jnp.log(jnp.maximum(der_den, 1e-12)))

    outside = jnp.logical_or(xq < left, xq > right)
    return jnp.where(outside, xq, y), jnp.where(outside, 0.0, logdet)


def _fused_kernel(x_ref, cond_ref, w0_ref, b0_ref, w1_ref, b1_ref,
                  w2_ref, b2_ref, y_ref, ld_ref):
    f32 = jnp.float32
    c = cond_ref[:]
    h = jnp.maximum(jnp.dot(c, w0_ref[:], preferred_element_type=f32) + b0_ref[:], 0.0)
    h = jnp.maximum(jnp.dot(h, w1_ref[:], preferred_element_type=f32) + b1_ref[:], 0.0)
    p = jnp.dot(h, w2_ref[:], preferred_element_type=f32) + b2_ref[:]

    r_i = lax.broadcasted_iota(jnp.int32, (_NUM_BINS, _NUM_BINS), 0)
    c_i = lax.broadcasted_iota(jnp.int32, (_NUM_BINS, _NUM_BINS), 1)
    tri = (r_i <= c_i).astype(f32)

    y, ld = _spline_chunk(p, x_ref[:], tri)
    y_ref[:] = y
    ld_ref[:] = ld


@jax.jit
def kernel(x, cond, W0, b0, W1, b1, W2, b2):
    x2 = x.reshape(_B, 1)
    W2p = jnp.pad(W2, ((0, 0), (0, _OUT_PAD - _OUT_DIM)))
    b2p = jnp.pad(b2, (0, _OUT_PAD - _OUT_DIM)).reshape(1, _OUT_PAD)
    b0r = b0.reshape(1, _HIDDEN)
    b1r = b1.reshape(1, _HIDDEN)
    grid = (_NSTEP,)
    y2, ld2 = pl.pallas_call(
        _fused_kernel,
        grid=grid,
        in_specs=[
            pl.BlockSpec((_TILE, 1), lambda i: (i, 0)),
            pl.BlockSpec((_TILE, _COND_DIM), lambda i: (i, 0)),
            pl.BlockSpec((_COND_DIM, _HIDDEN), lambda i: (0, 0)),
            pl.BlockSpec((1, _HIDDEN), lambda i: (0, 0)),
            pl.BlockSpec((_HIDDEN, _HIDDEN), lambda i: (0, 0)),
            pl.BlockSpec((1, _HIDDEN), lambda i: (0, 0)),
            pl.BlockSpec((_HIDDEN, _OUT_PAD), lambda i: (0, 0)),
            pl.BlockSpec((1, _OUT_PAD), lambda i: (0, 0)),
        ],
        out_specs=[
            pl.BlockSpec((_TILE, 1), lambda i: (i, 0)),
            pl.BlockSpec((_TILE, 1), lambda i: (i, 0)),
        ],
        out_shape=[
            jax.ShapeDtypeStruct((_B, 1), jnp.float32),
            jax.ShapeDtypeStruct((_B, 1), jnp.float32),
        ],
        compiler_params=pltpu.CompilerParams(
            dimension_semantics=("arbitrary",),
        ),
    )(x2, cond, W0, b0r, W1, b1r, W2p, b2p)
    return y2.reshape(_B), ld2.reshape(_B)
